# baseline (speedup 1.0000x reference)
import jax
import jax.numpy as jnp
from jax import lax
from jax.experimental import pallas as pl
from jax.experimental.pallas import tpu as pltpu
from jax.experimental.pallas import tpu_sc as plsc

_BATCH = 16384
_FEAT = 64
_CLASSES = 100000
_SPLIT = 49920
_H1 = _CLASSES - _SPLIT
_LANES = 16
_NC = 2
_NS = 16
_NW = _NC * _NS
_DPW = _FEAT // _NW
_TCHUNK = 4096
_NTC = _BATCH // _TCHUNK
_UNROLL = 4
_TSTAGE = _BATCH // _NS


def _tec_body(ft_hbm, tgt_hbm, ct_hbm, out_hbm,
              cbuf_a, cbuf_b, frow_v, tgt_v, part_v, tgt_sh,
              sem_a, sem_b, fsem, tsem):
    c = lax.axis_index("c")
    s = lax.axis_index("s")
    wid = s * _NC + c

    pltpu.sync_copy(tgt_hbm.at[pl.ds(s * _TSTAGE, _TSTAGE)],
                    tgt_sh.at[pl.ds(s * _TSTAGE, _TSTAGE)])
    plsc.subcore_barrier()

    bufs = (cbuf_a, cbuf_b)
    sems = (sem_a, sem_b)

    def chalf_copy(u):
        j, h = divmod(u, 2)
        d = wid * _DPW + j
        if h == 0:
            return pltpu.async_copy(
                ct_hbm.at[pl.ds(d, 1), pl.ds(0, _SPLIT)],
                bufs[u % 2].at[pl.ds(0, 1), pl.ds(0, _SPLIT)], sems[u % 2])
        return pltpu.async_copy(
            ct_hbm.at[pl.ds(d, 1), pl.ds(_SPLIT, _H1)],
            bufs[u % 2], sems[u % 2])

    acc = (jnp.zeros((_LANES,), jnp.float32),) * _UNROLL
    cps = [chalf_copy(0), chalf_copy(1)]
    cp_f = pltpu.async_copy(ft_hbm.at[wid * _DPW], frow_v, fsem)
    for u in range(2 * _DPW):
        j, h = divmod(u, 2)
        cps[u % 2].wait()
        if u + 2 < 2 * _DPW:
            cps[u % 2] = chalf_copy(u + 2)
        if u == 0:
            cp_f.wait()
        if h == 0 and u > 0:
            pltpu.sync_copy(ft_hbm.at[wid * _DPW + j], frow_v)
        cbuf = bufs[u % 2]
        cp_t = pltpu.async_copy(tgt_sh.at[pl.ds(0, _TCHUNK)], tgt_v.at[0],
                                tsem)
        for t in range(_NTC):
            cp_t.wait()
            if t + 1 < _NTC:
                cp_t = pltpu.async_copy(
                    tgt_sh.at[pl.ds((t + 1) * _TCHUNK, _TCHUNK)],
                    tgt_v.at[(t + 1) % 2], tsem)

            def step(k, a, _t=t, _h=h, _cbuf=cbuf):
                i0 = k * (_LANES * _UNROLL)
                res = []
                for v in range(_UNROLL):
                    off = i0 + v * _LANES
                    tv = tgt_v[_t % 2, pl.ds(off, _LANES)]
                    fv = frow_v[pl.ds(_t * _TCHUNK + off, _LANES)]
                    if _h == 0:
                        mask = tv < _SPLIT
                        idx = tv
                    else:
                        mask = tv >= _SPLIT
                        idx = tv - _SPLIT
                    cv = plsc.load_gather(_cbuf.at[0], [idx], mask=mask)
                    dv = jnp.where(mask, fv - cv, 0.0)
                    res.append(a[v] + dv * dv)
                return tuple(res)

            acc = lax.fori_loop(0, _TCHUNK // (_LANES * _UNROLL), step, acc)

    part = ((acc[0] + acc[1]) + (acc[2] + acc[3])) * (0.5 / _BATCH)
    part_v[...] = part
    pltpu.sync_copy(part_v, out_hbm.at[wid])


def _center_loss(features_t, targets, centers_t):
    mesh = plsc.VectorSubcoreMesh(core_axis_name="c", subcore_axis_name="s")
    run = pl.kernel(
        _tec_body,
        mesh=mesh,
        out_type=jax.ShapeDtypeStruct((_NW, _LANES), jnp.float32),
        scratch_types=[
            pltpu.VMEM((1, _H1), jnp.float32),
            pltpu.VMEM((1, _H1), jnp.float32),
            pltpu.VMEM((_BATCH,), jnp.float32),
            pltpu.VMEM((2, _TCHUNK), jnp.int32),
            pltpu.VMEM((_LANES,), jnp.float32),
            pltpu.VMEM_SHARED((_BATCH,), jnp.int32),
            pltpu.SemaphoreType.DMA,
            pltpu.SemaphoreType.DMA,
            pltpu.SemaphoreType.DMA,
            pltpu.SemaphoreType.DMA,
        ],
        compiler_params=pltpu.CompilerParams(needs_layout_passes=False),
    )
    parts = run(features_t, targets, centers_t)
    return jnp.sum(parts)


def kernel(features, targets, centers):
    return _center_loss(features.T, targets.astype(jnp.int32), centers.T)


# speedup vs baseline: 1.0686x; 1.0686x over previous
import jax
import jax.numpy as jnp
from jax import lax
from jax.experimental import pallas as pl
from jax.experimental.pallas import tpu as pltpu
from jax.experimental.pallas import tpu_sc as plsc

_BATCH = 16384
_FEAT = 64
_CLASSES = 100000
_LANES = 16
_NC = 2
_NS = 16
_NW = _NC * _NS
_DPW = _FEAT // _NW
_TCHUNK = 4096
_NTC = _BATCH // _TCHUNK
_UNROLL = 4
_NSEG = 4
_SEG = 25088
_TSTAGE = _BATCH // _NS


def _tec_body(ft_hbm, tgt_hbm, ct_hbm, out_hbm,
              crow_v, frow_v, tgt_v, part_v, tgt_sh, sems, fsem, tsem):
    c = lax.axis_index("c")
    s = lax.axis_index("s")
    wid = s * _NC + c

    pltpu.sync_copy(tgt_hbm.at[pl.ds(s * _TSTAGE, _TSTAGE)],
                    tgt_sh.at[pl.ds(s * _TSTAGE, _TSTAGE)])
    plsc.subcore_barrier()

    acc = (jnp.zeros((_LANES,), jnp.float32),) * _UNROLL
    for j in range(_DPW):
        d = wid * _DPW + j
        cps = []
        for g in range(_NSEG):
            lo = g * _SEG
            ln = min(_SEG, _CLASSES - lo)
            cps.append(pltpu.async_copy(
                ct_hbm.at[pl.ds(d, 1), pl.ds(lo, ln)],
                crow_v.at[pl.ds(0, 1), pl.ds(lo, ln)], sems[g]))
        cp_f = pltpu.async_copy(ft_hbm.at[wid * _DPW + j], frow_v, fsem)
        cp_t = pltpu.async_copy(tgt_sh.at[pl.ds(0, _TCHUNK)], tgt_v.at[0],
                                tsem)
        for cp in cps:
            cp.wait()
        cp_f.wait()
        for t in range(_NTC):
            cp_t.wait()
            if t + 1 < _NTC:
                cp_t = pltpu.async_copy(
                    tgt_sh.at[pl.ds((t + 1) * _TCHUNK, _TCHUNK)],
                    tgt_v.at[(t + 1) % 2], tsem)

            def step(k, a, _t=t):
                i0 = k * (_LANES * _UNROLL)
                res = []
                for v in range(_UNROLL):
                    off = i0 + v * _LANES
                    tv = tgt_v[_t % 2, pl.ds(off, _LANES)]
                    fv = frow_v[pl.ds(_t * _TCHUNK + off, _LANES)]
                    cv = plsc.load_gather(crow_v.at[0], [tv])
                    dv = fv - cv
                    res.append(a[v] + dv * dv)
                return tuple(res)

            acc = lax.fori_loop(0, _TCHUNK // (_LANES * _UNROLL), step, acc)

    part = ((acc[0] + acc[1]) + (acc[2] + acc[3])) * (0.5 / _BATCH)
    part_v[...] = part
    pltpu.sync_copy(part_v, out_hbm.at[wid])


def _center_loss(features_t, targets, centers_t):
    mesh = plsc.VectorSubcoreMesh(core_axis_name="c", subcore_axis_name="s")
    run = pl.kernel(
        _tec_body,
        mesh=mesh,
        out_type=jax.ShapeDtypeStruct((_NW, _LANES), jnp.float32),
        scratch_types=[
            pltpu.VMEM((1, _CLASSES), jnp.float32),
            pltpu.VMEM((_BATCH,), jnp.float32),
            pltpu.VMEM((2, _TCHUNK), jnp.int32),
            pltpu.VMEM((_LANES,), jnp.float32),
            pltpu.VMEM_SHARED((_BATCH,), jnp.int32),
            [pltpu.SemaphoreType.DMA] * _NSEG,
            pltpu.SemaphoreType.DMA,
            pltpu.SemaphoreType.DMA,
        ],
        compiler_params=pltpu.CompilerParams(needs_layout_passes=False),
    )
    parts = run(features_t, targets, centers_t)
    return jnp.sum(parts)


def kernel(features, targets, centers):
    return _center_loss(features.T, targets.astype(jnp.int32), centers.T)
